# Initial kernel scaffold; baseline (speedup 1.0000x reference)
import jax
import jax.numpy as jnp
from jax.experimental import pallas as pl


def kernel(x, We0, be0, We1, be1, We2, be2, Wd0, bd0, Wd1, bd1, Wd2, bd2, codebooks):
    raise NotImplementedError("write your pallas kernel here")


# speedup vs baseline: 1.0550x; 1.0550x over previous
import functools

import jax
import jax.numpy as jnp
from jax.experimental import pallas as pl
from jax.experimental.pallas import tpu as pltpu

_BATCH = 8192
_BT = 512
_K = 1024
_E = 256
_L = 4

_PREC = jax.lax.Precision.DEFAULT
_PREC_HI = jax.lax.Precision.HIGHEST


def _dot(a, b, prec=_PREC):
    return jax.lax.dot_general(a, b, (((1,), (0,)), ((), ())), precision=prec,
                               preferred_element_type=jnp.float32)


def _dot_t(a, b, prec=_PREC):
    return jax.lax.dot_general(a, b, (((1,), (1,)), ((), ())), precision=prec,
                               preferred_element_type=jnp.float32)


def _body(x_ref, We0_ref, be0_ref, We1_ref, be1_ref, We2_ref, be2_ref,
          Wd0_ref, bd0_ref, Wd1_ref, bd1_ref, Wd2_ref, bd2_ref, cb_ref,
          y_ref, loss_ref):
    x = x_ref[...]
    h = jnp.maximum(_dot(x, We0_ref[...]) + be0_ref[...], 0.0)
    h = jnp.maximum(_dot(h, We1_ref[...]) + be1_ref[...], 0.0)
    z = _dot(h, We2_ref[...]) + be2_ref[...]

    residual = z
    x_q = jnp.zeros_like(z)
    loss_sum = jnp.float32(0.0)
    iota = jax.lax.broadcasted_iota(jnp.int32, (_BT, _K), 1)
    for l in range(_L):
        cb = cb_ref[l]
        cb_sq = jnp.sum(cb * cb, axis=1)
        r_sq = jnp.sum(residual * residual, axis=1, keepdims=True)
        d = (r_sq - 2.0 * _dot_t(residual, cb)) + cb_sq[None, :]
        m = jnp.min(d, axis=1, keepdims=True)
        idx = jnp.min(jnp.where(d == m, iota, _K), axis=1)
        onehot = (iota == idx[:, None]).astype(jnp.float32)
        q = _dot(onehot, cb, prec=_PREC_HI)
        diff = residual - q
        loss_sum = loss_sum + jnp.sum(diff * diff)
        residual = diff
        x_q = x_q + q

    h = jnp.maximum(_dot(x_q, Wd0_ref[...]) + bd0_ref[...], 0.0)
    h = jnp.maximum(_dot(h, Wd1_ref[...]) + bd1_ref[...], 0.0)
    y_ref[...] = _dot(h, Wd2_ref[...]) + bd2_ref[...]

    @pl.when(pl.program_id(0) == 0)
    def _init():
        loss_ref[...] = jnp.zeros_like(loss_ref)

    scale = 1.25 / (_L * _BATCH * _E)
    loss_ref[...] += jnp.reshape(loss_sum * scale, (1, 1))


def kernel(x, We0, be0, We1, be1, We2, be2, Wd0, bd0, Wd1, bd1, Wd2, bd2,
           codebooks):
    nb = _BATCH // _BT
    full = lambda shape: pl.BlockSpec(shape, lambda i: (0,) * len(shape))
    row = lambda n: pl.BlockSpec((1, n), lambda i: (0, 0))
    y, loss = pl.pallas_call(
        _body,
        grid=(nb,),
        in_specs=[
            pl.BlockSpec((_BT, 768), lambda i: (i, 0)),
            full((768, 2048)), row(2048),
            full((2048, 1024)), row(1024),
            full((1024, 256)), row(256),
            full((256, 1024)), row(1024),
            full((1024, 2048)), row(2048),
            full((2048, 768)), row(768),
            full((_L, _K, _E)),
        ],
        out_specs=[
            pl.BlockSpec((_BT, 768), lambda i: (i, 0)),
            pl.BlockSpec((1, 1), lambda i: (0, 0)),
        ],
        out_shape=[
            jax.ShapeDtypeStruct((_BATCH, 768), jnp.float32),
            jax.ShapeDtypeStruct((1, 1), jnp.float32),
        ],
        compiler_params=pltpu.CompilerParams(
            dimension_semantics=("arbitrary",),
            vmem_limit_bytes=110 * 1024 * 1024,
        ),
    )(x, We0, be0.reshape(1, -1), We1, be1.reshape(1, -1),
      We2, be2.reshape(1, -1), Wd0, bd0.reshape(1, -1),
      Wd1, bd1.reshape(1, -1), Wd2, bd2.reshape(1, -1), codebooks)
    return (y, loss[0, 0])
